# Initial kernel scaffold; baseline (speedup 1.0000x reference)
import jax
import jax.numpy as jnp
from jax.experimental import pallas as pl


def kernel(x, edge_index, edge_attr, batch, We1a, be1a, We1b, be1b, root1, bias1, g1, bt1, We2a, be2a, We2b, be2b, root2, bias2, g2, bt2, Wout, bout):
    raise NotImplementedError("write your pallas kernel here")


# speedup vs baseline: 2.0645x; 2.0645x over previous
import functools

import jax
import jax.numpy as jnp
from jax import lax
from jax.experimental import pallas as pl
from jax.experimental.pallas import tpu as pltpu
from jax.experimental.pallas import tpu_sc as plsc

N = 10000
E = 160000
IN = 128
H = 8
ED = 16
G = 16
EMB = 64
HN = H * IN
HH = H * H
D = 16

NC = 2
NS = 16
NW = NC * NS
CHUNK = 128
NCH = E // CHUNK
ZSTR = 1000
NZT = N // ZSTR

_MESH = plsc.VectorSubcoreMesh(core_axis_name="c", subcore_axis_name="s")
_SC_PARAMS = pltpu.CompilerParams(use_tc_tiling_on_sc=False)


def _sc_gather(table, idx3):
    n, d = table.shape

    @functools.partial(
        pl.kernel,
        out_type=jax.ShapeDtypeStruct((E, d), jnp.float32),
        mesh=_MESH,
        compiler_params=_SC_PARAMS,
        scratch_types=[
            pltpu.VMEM((1, CHUNK), jnp.int32),
            pltpu.VMEM((CHUNK, d), jnp.float32),
            pltpu.SemaphoreType.DMA,
        ],
    )
    def gk(tab_hbm, idx_hbm, out_hbm, idx_v, buf, sem):
        wid = lax.axis_index("s") * NC + lax.axis_index("c")
        nchw = (NCH - wid + NW - 1) // NW

        def body(t, carry):
            c = wid + t * NW
            pltpu.sync_copy(idx_hbm.at[c], idx_v)
            pltpu.async_copy(tab_hbm.at[idx_v.at[0]], buf, sem).wait()
            pltpu.sync_copy(buf, out_hbm.at[pl.ds(c * CHUNK, CHUNK)])
            return carry

        lax.fori_loop(0, nchw, body, 0)

    return gk(table, idx3)


def _sc_scatter_add(msg, idx3, zrows):
    @functools.partial(
        pl.kernel,
        out_type=jax.ShapeDtypeStruct((NC, N, D), jnp.float32),
        mesh=_MESH,
        compiler_params=_SC_PARAMS,
        scratch_types=[
            pltpu.VMEM((1, CHUNK), jnp.int32),
            pltpu.VMEM((CHUNK, D), jnp.float32),
            pltpu.VMEM((ZSTR, D), jnp.float32),
            pltpu.VMEM_SHARED((N, D), jnp.float32),
        ],
    )
    def sk(msg_hbm, idx_hbm, z_hbm, out_hbm, idx_v, msg_v, tmp_v, acc_sh):
        cid = lax.axis_index("c")
        sid = lax.axis_index("s")
        wid = sid * NC + cid
        nchw = (NCH - wid + NW - 1) // NW

        @pl.when(sid < NZT)
        def _zero():
            pltpu.sync_copy(z_hbm.at[pl.ds(sid * ZSTR, ZSTR)], tmp_v)
            pltpu.sync_copy(tmp_v, acc_sh.at[pl.ds(sid * ZSTR, ZSTR)])

        plsc.subcore_barrier()

        def body(t, carry):
            c = wid + t * NW
            pltpu.sync_copy(idx_hbm.at[c], idx_v)
            pltpu.sync_copy(msg_hbm.at[pl.ds(c * CHUNK, CHUNK)], msg_v)
            pltpu.sync_copy(msg_v, acc_sh.at[idx_v.at[0]], add=True)
            return carry

        lax.fori_loop(0, nchw, body, 0)
        plsc.subcore_barrier()

        @pl.when(sid < NZT)
        def _out():
            pltpu.sync_copy(acc_sh.at[pl.ds(sid * ZSTR, ZSTR)], tmp_v)
            pltpu.sync_copy(tmp_v, out_hbm.at[cid].at[pl.ds(sid * ZSTR, ZSTR)])

    return sk(msg, idx3, zrows)


_EB = 1000
_NB = 1000


def _edge_l1(ea, xg, wa, ba, wbp_bf, bbp, p1):
    grid = (E // _EB,)

    def body(ea_ref, xg_ref, wa_ref, ba_ref, wb_ref, bb_ref, p_ref, out_ref):
        h = jnp.dot(ea_ref[...], wa_ref[...], preferred_element_type=jnp.float32)
        h = jnp.maximum(h + ba_ref[...], 0.0)
        w = jnp.dot(h.astype(jnp.bfloat16), wb_ref[...],
                    preferred_element_type=jnp.float32) + bb_ref[...]
        xt = jnp.concatenate([xg_ref[...]] * H, axis=1)
        m = (w * xt).astype(jnp.bfloat16)
        msg = jnp.dot(m, p_ref[...], preferred_element_type=jnp.float32)
        out_ref[...] = jnp.concatenate([msg, jnp.zeros_like(msg)], axis=1)

    return pl.pallas_call(
        body,
        grid=grid,
        in_specs=[
            pl.BlockSpec((_EB, ED), lambda i: (i, 0)),
            pl.BlockSpec((_EB, IN), lambda i: (i, 0)),
            pl.BlockSpec((ED, HN), lambda i: (0, 0)),
            pl.BlockSpec((1, HN), lambda i: (0, 0)),
            pl.BlockSpec((HN, HN), lambda i: (0, 0)),
            pl.BlockSpec((1, HN), lambda i: (0, 0)),
            pl.BlockSpec((HN, H), lambda i: (0, 0)),
        ],
        out_specs=pl.BlockSpec((_EB, D), lambda i: (i, 0)),
        out_shape=jax.ShapeDtypeStruct((E, D), jnp.float32),
    )(ea, xg, wa, ba, wbp_bf, bbp, p1)


def _edge_l2(ea, o1g, wa, ba, wbp, bbp, p2):
    grid = (E // _EB,)

    def body(ea_ref, o1_ref, wa_ref, ba_ref, wb_ref, bb_ref, p_ref, out_ref):
        h = jnp.dot(ea_ref[...], wa_ref[...], preferred_element_type=jnp.float32)
        h = jnp.maximum(h + ba_ref[...], 0.0)
        w = jnp.dot(h, wb_ref[...], preferred_element_type=jnp.float32) + bb_ref[...]
        o1 = o1_ref[...][:, :H]
        xt = jnp.concatenate([o1] * H, axis=1)
        msg = jnp.dot(w * xt, p_ref[...], preferred_element_type=jnp.float32)
        out_ref[...] = jnp.concatenate([msg, jnp.zeros_like(msg)], axis=1)

    return pl.pallas_call(
        body,
        grid=grid,
        in_specs=[
            pl.BlockSpec((_EB, ED), lambda i: (i, 0)),
            pl.BlockSpec((_EB, D), lambda i: (i, 0)),
            pl.BlockSpec((ED, HH), lambda i: (0, 0)),
            pl.BlockSpec((1, HH), lambda i: (0, 0)),
            pl.BlockSpec((HH, HH), lambda i: (0, 0)),
            pl.BlockSpec((1, HH), lambda i: (0, 0)),
            pl.BlockSpec((HH, H), lambda i: (0, 0)),
        ],
        out_specs=pl.BlockSpec((_EB, D), lambda i: (i, 0)),
        out_shape=jax.ShapeDtypeStruct((E, D), jnp.float32),
    )(ea, o1g, wa, ba, wbp, bbp, p2)


def _ln_lrelu(a, g, bt):
    m = jnp.mean(a, axis=1, keepdims=True)
    v = jnp.mean((a - m) ** 2, axis=1, keepdims=True)
    y = (a - m) * lax.rsqrt(v + 1e-5) * g + bt
    return jnp.where(y >= 0, y, 0.1 * y)


def _node1(part, x, root, bias, g, bt):
    grid = (N // _NB,)

    def body(p_ref, x_ref, r_ref, b_ref, g_ref, bt_ref, out_ref):
        a = p_ref[0, :, :H] + p_ref[1, :, :H]
        a = a + jnp.dot(x_ref[...], r_ref[...],
                        preferred_element_type=jnp.float32) + b_ref[...]
        y = _ln_lrelu(a, g_ref[...], bt_ref[...])
        out_ref[...] = jnp.concatenate([y, jnp.zeros_like(y)], axis=1)

    return pl.pallas_call(
        body,
        grid=grid,
        in_specs=[
            pl.BlockSpec((NC, _NB, D), lambda i: (0, i, 0)),
            pl.BlockSpec((_NB, IN), lambda i: (i, 0)),
            pl.BlockSpec((IN, H), lambda i: (0, 0)),
            pl.BlockSpec((1, H), lambda i: (0, 0)),
            pl.BlockSpec((1, H), lambda i: (0, 0)),
            pl.BlockSpec((1, H), lambda i: (0, 0)),
        ],
        out_specs=pl.BlockSpec((_NB, D), lambda i: (i, 0)),
        out_shape=jax.ShapeDtypeStruct((N, D), jnp.float32),
    )(part, x, root, bias, g, bt)


def _node2_pool(part, out1, ohT, oh, root, bias, g, bt, wo, bo):
    grid = N // _NB

    def body(p_ref, o1_ref, ohT_ref, oh_ref, r_ref, b_ref, g_ref, bt_ref,
             wo_ref, bo_ref, out_ref, mx_s, sm_s, cnt_s):
        i = pl.program_id(0)

        @pl.when(i == 0)
        def _init():
            mx_s[...] = jnp.full((G, H), -jnp.inf, jnp.float32)
            sm_s[...] = jnp.zeros((G, H), jnp.float32)
            cnt_s[...] = jnp.zeros((G, 1), jnp.float32)

        o1 = o1_ref[...][:, :H]
        a = p_ref[0, :, :H] + p_ref[1, :, :H]
        a = a + jnp.dot(o1, r_ref[...],
                        preferred_element_type=jnp.float32) + b_ref[...]
        y = _ln_lrelu(a, g_ref[...], bt_ref[...])

        ohT_blk = ohT_ref[0]
        sm_s[...] += jnp.dot(ohT_blk, y, preferred_element_type=jnp.float32)
        cnt_s[...] += jnp.dot(ohT_blk, jnp.ones((_NB, 1), jnp.float32),
                              preferred_element_type=jnp.float32)
        oh_blk = oh_ref[...]
        for gi in range(G):
            mask = oh_blk[:, gi:gi + 1] > 0.5
            cand = jnp.max(jnp.where(mask, y, -jnp.inf), axis=0, keepdims=True)
            mx_s[pl.ds(gi, 1), :] = jnp.maximum(mx_s[pl.ds(gi, 1), :], cand)

        @pl.when(i == grid - 1)
        def _fin():
            mean = sm_s[...] / jnp.maximum(cnt_s[...], 1.0)
            pooled = jnp.concatenate([mx_s[...], mean], axis=1)
            out_ref[...] = jnp.dot(pooled, wo_ref[...],
                                   preferred_element_type=jnp.float32) + bo_ref[...]

    return pl.pallas_call(
        body,
        grid=(grid,),
        in_specs=[
            pl.BlockSpec((NC, _NB, D), lambda i: (0, i, 0)),
            pl.BlockSpec((_NB, D), lambda i: (i, 0)),
            pl.BlockSpec((1, G, _NB), lambda i: (i, 0, 0)),
            pl.BlockSpec((_NB, G), lambda i: (i, 0)),
            pl.BlockSpec((H, H), lambda i: (0, 0)),
            pl.BlockSpec((1, H), lambda i: (0, 0)),
            pl.BlockSpec((1, H), lambda i: (0, 0)),
            pl.BlockSpec((1, H), lambda i: (0, 0)),
            pl.BlockSpec((2 * H, EMB), lambda i: (0, 0)),
            pl.BlockSpec((1, EMB), lambda i: (0, 0)),
        ],
        out_specs=pl.BlockSpec((G, EMB), lambda i: (0, 0)),
        out_shape=jax.ShapeDtypeStruct((G, EMB), jnp.float32),
        scratch_shapes=[
            pltpu.VMEM((G, H), jnp.float32),
            pltpu.VMEM((G, H), jnp.float32),
            pltpu.VMEM((G, 1), jnp.float32),
        ],
    )(part, out1, ohT, oh, root, bias, g, bt, wo, bo)


def kernel(x, edge_index, edge_attr, batch, We1a, be1a, We1b, be1b, root1,
           bias1, g1, bt1, We2a, be2a, We2b, be2b, root2, bias2, g2, bt2,
           Wout, bout):
    src3 = edge_index[0].reshape(NCH, 1, CHUNK)
    dst3 = edge_index[1].reshape(NCH, 1, CHUNK)

    wb1p = We1b.reshape(HN, IN, H).transpose(0, 2, 1).reshape(HN, HN)
    bb1p = be1b.reshape(IN, H).T.reshape(1, HN)
    wb2p = We2b.reshape(HH, H, H).transpose(0, 2, 1).reshape(HH, HH)
    bb2p = be2b.reshape(H, H).T.reshape(1, HH)
    p1 = jnp.kron(jnp.eye(H, dtype=jnp.bfloat16), jnp.ones((IN, 1), jnp.bfloat16))
    p2 = jnp.kron(jnp.eye(H, dtype=jnp.float32), jnp.ones((H, 1), jnp.float32))
    oh = (batch[:, None] == jnp.arange(G, dtype=batch.dtype)[None, :])
    oh = oh.astype(jnp.float32)
    ohT = oh.T.reshape(G, N // _NB, _NB).transpose(1, 0, 2)
    zrows = jnp.zeros((N, D), jnp.float32)

    xg = _sc_gather(x, src3)
    msg1 = _edge_l1(edge_attr, xg, We1a, be1a.reshape(1, HN),
                    wb1p.astype(jnp.bfloat16), bb1p, p1)
    part1 = _sc_scatter_add(msg1, dst3, zrows)
    out1 = _node1(part1, x, root1, bias1.reshape(1, H),
                  g1.reshape(1, H), bt1.reshape(1, H))

    o1g = _sc_gather(out1, src3)
    msg2 = _edge_l2(edge_attr, o1g, We2a, be2a.reshape(1, HH),
                    wb2p, bb2p, p2)
    part2 = _sc_scatter_add(msg2, dst3, zrows)

    return _node2_pool(part2, out1, ohT, oh, root2, bias2.reshape(1, H),
                       g2.reshape(1, H), bt2.reshape(1, H), Wout,
                       bout.reshape(1, EMB))
